# baseline (speedup 1.0000x reference)
import functools

import jax
import jax.numpy as jnp
from jax import lax
from jax.experimental import pallas as pl
from jax.experimental.pallas import tpu as pltpu
from jax.experimental.pallas import tpu_sc as plsc

E = 160000
LE = 320000
EMB = 128
NRAD = 6
NSPH = 7
NBIL = 8

NC = 2
NS = 16
LANES = 16
NW = NC * NS

W1 = LE // NW
CH1 = 80
NCH1 = W1 // CH1

SH2 = LE // NS
CSHIFT = 13
CROWS = 1 << CSHIFT
NCHUNK2 = 20
EPAD = NCHUNK2 * CROWS
SB_ROWS = CROWS + 8
TRASH = CROWS
SELR = 158
SELC = 128
TROWS = CROWS // NS
DSTG = 2000
NSTG = SH2 // DSTG
SGRP = DSTG // LANES


def _nr_rsqrt(s):
  i = lax.bitcast_convert_type(s, jnp.int32)
  i = jnp.int32(0x5F3759DF) - (i >> 1)
  t = lax.bitcast_convert_type(i, jnp.float32)
  for _ in range(3):
    t = t * (jnp.float32(1.5) - jnp.float32(0.5) * s * t * t)
  return t


def _sc_gather_body(ox_hbm, oy_hbm, oz_hbm, m_hbm, src_hbm, dst_hbm,
                    msrc_hbm, c_hbm,
                    idxs_a, idxd_a, mrows_a, xyz_a,
                    idxs_b, idxd_b, mrows_b, xyz_b,
                    c_v, semm_a, semo_a, semm_b, semo_b):
  cid = lax.axis_index("c")
  sid = lax.axis_index("s")
  wid = sid * NC + cid
  wbase = wid * W1
  bufs_a = (idxs_a, idxd_a, mrows_a, xyz_a, semm_a, semo_a)
  bufs_b = (idxs_b, idxd_b, mrows_b, xyz_b, semm_b, semo_b)

  def o_copies(idxs_v, idxd_v, xyz_v, semo):
    return [
        pltpu.make_async_copy(ox_hbm.at[idxs_v], xyz_v.at[0], semo),
        pltpu.make_async_copy(oy_hbm.at[idxs_v], xyz_v.at[1], semo),
        pltpu.make_async_copy(oz_hbm.at[idxs_v], xyz_v.at[2], semo),
        pltpu.make_async_copy(ox_hbm.at[idxd_v], xyz_v.at[3], semo),
        pltpu.make_async_copy(oy_hbm.at[idxd_v], xyz_v.at[4], semo),
        pltpu.make_async_copy(oz_hbm.at[idxd_v], xyz_v.at[5], semo),
    ]

  def fire(k, bufs):
    idxs_v, idxd_v, mrows_v, xyz_v, semm, semo = bufs
    off = wbase + k * CH1
    pltpu.sync_copy(src_hbm.at[pl.ds(off, CH1)], idxs_v)
    pltpu.sync_copy(dst_hbm.at[pl.ds(off, CH1)], idxd_v)
    pltpu.async_copy(m_hbm.at[idxs_v], mrows_v, semm)
    for cp in o_copies(idxs_v, idxd_v, xyz_v, semo):
      cp.start()

  def drain(k, bufs):
    idxs_v, idxd_v, mrows_v, xyz_v, semm, semo = bufs
    off = wbase + k * CH1
    for cp in o_copies(idxs_v, idxd_v, xyz_v, semo):
      cp.wait()
    for g in range(CH1 // LANES):
      sl = pl.ds(g * LANES, LANES)
      x1, y1, z1 = xyz_v[0, sl], xyz_v[1, sl], xyz_v[2, sl]
      x2, y2, z2 = xyz_v[3, sl], xyz_v[4, sl], xyz_v[5, sl]
      dot = x1 * x2 + y1 * y2 + z1 * z2
      cx = y1 * z2 - z1 * y2
      cy = z1 * x2 - x1 * z2
      cz = x1 * y2 - y1 * x2
      s = dot * dot + cx * cx + cy * cy + cz * cz
      c_v[sl] = dot * _nr_rsqrt(s)
    pltpu.make_async_copy(m_hbm.at[idxs_v], mrows_v, semm).wait()
    pltpu.sync_copy(mrows_v, msrc_hbm.at[pl.ds(off, CH1)])
    pltpu.sync_copy(c_v, c_hbm.at[pl.ds(off, CH1)])

  fire(0, bufs_a)

  def pair(i, carry):
    k0 = i * 2
    fire(k0 + 1, bufs_b)
    drain(k0, bufs_a)
    fire(k0 + 2, bufs_a)
    drain(k0 + 1, bufs_b)
    return carry

  lax.fori_loop(0, NCH1 // 2, pair, 0)
  drain(NCH1 - 1, bufs_a)


def _make_sc_gather():
  mesh = plsc.VectorSubcoreMesh(
      core_axis_name="c", subcore_axis_name="s", num_cores=NC,
      num_subcores=NS)
  return pl.kernel(
      _sc_gather_body,
      out_type=[
          jax.ShapeDtypeStruct((LE, EMB), jnp.float32),
          jax.ShapeDtypeStruct((LE,), jnp.float32),
      ],
      mesh=mesh,
      compiler_params=pltpu.CompilerParams(needs_layout_passes=False),
      scratch_types=[
          pltpu.VMEM((CH1,), jnp.int32),
          pltpu.VMEM((CH1,), jnp.int32),
          pltpu.VMEM((CH1, EMB), jnp.float32),
          pltpu.VMEM((6, CH1), jnp.float32),
          pltpu.VMEM((CH1,), jnp.int32),
          pltpu.VMEM((CH1,), jnp.int32),
          pltpu.VMEM((CH1, EMB), jnp.float32),
          pltpu.VMEM((6, CH1), jnp.float32),
          pltpu.VMEM((CH1,), jnp.float32),
          pltpu.SemaphoreType.DMA,
          pltpu.SemaphoreType.DMA,
          pltpu.SemaphoreType.DMA,
          pltpu.SemaphoreType.DMA,
      ],
  )


def _sc_scatter_body(dst_hbm, xkj_hbm, zeros_hbm, mupd_hbm,
                     dstg0_v, dstg1_v, psel_v, rows0_v, rows1_v,
                     wselb0_v, dselb0_v, wselb1_v, dselb1_v,
                     sbuf, semg0, semg1, sema0, sema1, semd0, semd1):
  cid = lax.axis_index("c")
  sid = lax.axis_index("s")
  lane = lax.iota(jnp.int32, 16)
  c0 = lane * 0
  wbase = sid * SH2

  def stage_copy(s, buf, sem):
    return pltpu.make_async_copy(
        dst_hbm.at[pl.ds(wbase + s * DSTG, DSTG)], buf, sem)

  def chunk(k, carry):
    chunk_id = k * NC + cid
    lo = chunk_id * CROWS
    pltpu.sync_copy(zeros_hbm.at[pl.ds(sid * TROWS, TROWS)],
                    sbuf.at[pl.ds(sid * TROWS, TROWS)])
    plsc.subcore_barrier()

    def scan_stage(stgbuf, carry2):
      def grp(g, c2):
        offm1, pwv = c2
        dv = plsc.load_gather(stgbuf, [lane + g * LANES])
        mask = (dv >> CSHIFT) == chunk_id
        mv = mask.astype(jnp.int32)
        pos = plsc.cumsum(mv) + offm1
        packed = pwv | (dv & (CROWS - 1))
        plsc.store_scatter(psel_v, [pos >> 7, pos & 127], packed,
                           mask=mask)
        pc = plsc.all_reduce_population_count(mask)
        return (offm1 + pc, pwv + (LANES << CSHIFT))

      return lax.fori_loop(0, SGRP, grp, carry2)

    stage_copy(0, dstg0_v, semd0).start()

    def stagepair(t, carry2):
      s0 = t * 2
      stage_copy(s0 + 1, dstg1_v, semd1).start()
      stage_copy(s0, dstg0_v, semd0).wait()
      carry2 = scan_stage(dstg0_v, carry2)

      @pl.when(t < (NSTG // 2) - 1)
      def _prefetch_next():
        stage_copy(s0 + 2, dstg0_v, semd0).start()

      stage_copy(s0 + 1, dstg1_v, semd1).wait()
      return scan_stage(dstg1_v, carry2)

    offm1, _ = lax.fori_loop(
        0, NSTG // 2, stagepair, (c0 - 1, lane << CSHIFT))
    nsel = jnp.max(offm1) + 1
    nb = (nsel + (SELC - 1)) // SELC

    def unpack(b, wselb_v, dselb_v):
      for g in range(SELC // LANES):
        colv = lane + g * LANES
        fl = (b * SELC) + colv
        p = plsc.load_gather(psel_v, [fl >> 7, fl & 127])
        pm = fl >= nsel
        wv = jnp.where(pm, 0,
                       wbase + lax.shift_right_logical(p, CSHIFT))
        dv = jnp.where(pm, TRASH + (colv & 7), p & (CROWS - 1))
        plsc.store_scatter(wselb_v, [colv], wv)
        plsc.store_scatter(dselb_v, [colv], dv)

    def add0_copy():
      return pltpu.make_async_copy(rows0_v, sbuf.at[dselb0_v], sema0)

    def add1_copy():
      return pltpu.make_async_copy(rows1_v, sbuf.at[dselb1_v], sema1)

    def subpair(i, carry3):
      b0 = i * 2
      b1 = b0 + 1

      @pl.when(b0 < nb)
      def _fire0():
        @pl.when(b0 >= 2)
        def _reclaim0():
          add0_copy().wait()
        unpack(b0, wselb0_v, dselb0_v)
        pltpu.async_copy(xkj_hbm.at[wselb0_v], rows0_v, semg0)

      @pl.when(b1 < nb)
      def _fire1():
        @pl.when(b1 >= 2)
        def _reclaim1():
          add1_copy().wait()
        unpack(b1, wselb1_v, dselb1_v)
        pltpu.async_copy(xkj_hbm.at[wselb1_v], rows1_v, semg1)

      @pl.when(b0 < nb)
      def _drain0():
        pltpu.make_async_copy(xkj_hbm.at[wselb0_v], rows0_v, semg0).wait()
        pltpu.async_copy(rows0_v, sbuf.at[dselb0_v], sema0, add=True)

      @pl.when(b1 < nb)
      def _drain1():
        pltpu.make_async_copy(xkj_hbm.at[wselb1_v], rows1_v, semg1).wait()
        pltpu.async_copy(rows1_v, sbuf.at[dselb1_v], sema1, add=True)

      return carry3

    lax.fori_loop(0, (nb + 1) >> 1, subpair, 0)

    @pl.when(nb >= 1)
    def _final0():
      add0_copy().wait()

    @pl.when(nb >= 2)
    def _final1():
      add1_copy().wait()

    plsc.subcore_barrier()
    out0 = lo + sid * TROWS

    @pl.when(out0 + TROWS <= E)
    def _full_write():
      pltpu.sync_copy(sbuf.at[pl.ds(sid * TROWS, TROWS)],
                      mupd_hbm.at[pl.ds(out0, TROWS)])

    @pl.when(out0 == (E // TROWS) * TROWS)
    def _partial_write():
      pltpu.sync_copy(sbuf.at[pl.ds(sid * TROWS, E - (E // TROWS) * TROWS)],
                      mupd_hbm.at[pl.ds(out0, E - (E // TROWS) * TROWS)])

    return carry

  lax.fori_loop(0, NCHUNK2 // NC, chunk, 0)


def _make_sc_scatter():
  mesh = plsc.VectorSubcoreMesh(
      core_axis_name="c", subcore_axis_name="s", num_cores=NC,
      num_subcores=NS)
  return pl.kernel(
      _sc_scatter_body,
      out_type=jax.ShapeDtypeStruct((E, EMB), jnp.float32),
      mesh=mesh,
      compiler_params=pltpu.CompilerParams(needs_layout_passes=False),
      scratch_types=[
          pltpu.VMEM((DSTG,), jnp.int32),
          pltpu.VMEM((DSTG,), jnp.int32),
          pltpu.VMEM((SELR, 128), jnp.int32),
          pltpu.VMEM((SELC, EMB), jnp.float32),
          pltpu.VMEM((SELC, EMB), jnp.float32),
          pltpu.VMEM((SELC,), jnp.int32),
          pltpu.VMEM((SELC,), jnp.int32),
          pltpu.VMEM((SELC,), jnp.int32),
          pltpu.VMEM((SELC,), jnp.int32),
          pltpu.VMEM_SHARED((SB_ROWS, EMB), jnp.float32),
          pltpu.SemaphoreType.DMA,
          pltpu.SemaphoreType.DMA,
          pltpu.SemaphoreType.DMA,
          pltpu.SemaphoreType.DMA,
          pltpu.SemaphoreType.DMA,
          pltpu.SemaphoreType.DMA,
      ],
  )


BE = 640


def _w_body(rbf_ref, m_ref, wrbf_ref, wm_ref, bm_ref, w_ref):
  mt = jnp.dot(m_ref[...], wm_ref[...],
               preferred_element_type=jnp.float32) + bm_ref[...]
  mt = mt * jax.nn.sigmoid(mt)
  rp = jnp.dot(rbf_ref[...], wrbf_ref[...],
               preferred_element_type=jnp.float32)
  w_ref[...] = rp * mt


def _make_w_kernel():
  return pl.pallas_call(
      _w_body,
      grid=(E // BE,),
      in_specs=[
          pl.BlockSpec((BE, NRAD), lambda i: (i, 0)),
          pl.BlockSpec((BE, EMB), lambda i: (i, 0)),
          pl.BlockSpec((NRAD, EMB), lambda i: (0, 0)),
          pl.BlockSpec((EMB, EMB), lambda i: (0, 0)),
          pl.BlockSpec((EMB,), lambda i: (0,)),
      ],
      out_specs=pl.BlockSpec((BE, EMB), lambda i: (i, 0)),
      out_shape=jax.ShapeDtypeStruct((E, EMB), jnp.float32),
  )


BL = 512


def _xkj_body(c_ref, msrc_ref, wr_ref, w2_ref, xkj_ref):
  c = c_ref[...]
  ms = msrc_ref[...].astype(jnp.bfloat16)
  ts = [jnp.ones_like(c), c]
  for _ in range(2, NSPH):
    ts.append(jnp.float32(2.0) * c * ts[-1] - ts[-2])
  acc = None
  for j in range(NBIL):
    sj = ts[0] * wr_ref[0, j]
    for n in range(1, NSPH):
      sj = sj + ts[n] * wr_ref[n, j]
    sjb = sj.astype(jnp.bfloat16)
    p = jnp.dot(ms * sjb[:, None], w2_ref[j * EMB:(j + 1) * EMB, :],
                preferred_element_type=jnp.float32)
    acc = p if acc is None else acc + p
  xkj_ref[...] = acc


def _make_xkj_kernel():
  return pl.pallas_call(
      _xkj_body,
      grid=(LE // BL,),
      in_specs=[
          pl.BlockSpec((BL,), lambda i: (i,)),
          pl.BlockSpec((BL, EMB), lambda i: (i, 0)),
          pl.BlockSpec(memory_space=pltpu.SMEM),
          pl.BlockSpec((NBIL * EMB, EMB), lambda i: (0, 0)),
      ],
      out_specs=pl.BlockSpec((BL, EMB), lambda i: (i, 0)),
      out_shape=jax.ShapeDtypeStruct((LE, EMB), jnp.float32),
  )


def kernel(rbf, m, o, lg_src, lg_dst, W_rbf, W_sbf, W_m, b_m, W_bilin):
  ox, oy, oz = o[:, 0], o[:, 1], o[:, 2]
  wr = W_sbf.reshape(NSPH, NRAD, NBIL).sum(axis=1)
  w2 = W_bilin.transpose(1, 2, 0).reshape(NBIL * EMB, EMB).astype(jnp.bfloat16)
  zeros_c = jnp.zeros((CROWS, EMB), jnp.float32)

  m_src, c = _make_sc_gather()(ox, oy, oz, m, lg_src, lg_dst)
  w = _make_w_kernel()(rbf, m, W_rbf, W_m, b_m)
  x_kj = _make_xkj_kernel()(c, m_src, wr, w2)
  m_update = _make_sc_scatter()(lg_dst, x_kj, zeros_c)
  return (m_update, w)


# speedup vs baseline: 1.2785x; 1.2785x over previous
import functools

import jax
import jax.numpy as jnp
from jax import lax
from jax.experimental import pallas as pl
from jax.experimental.pallas import tpu as pltpu
from jax.experimental.pallas import tpu_sc as plsc

E = 160000
LE = 320000
EMB = 128
NRAD = 6
NSPH = 7
NBIL = 8

NC = 2
NS = 16
LANES = 16
NW = NC * NS

W1 = LE // NW
CH1 = 80
NCH1 = W1 // CH1

SH2 = LE // NS
CSHIFT = 13
CROWS = 1 << CSHIFT
NCHUNK2 = 20
EPAD = NCHUNK2 * CROWS
SB_ROWS = CROWS + 8
TRASH = CROWS
NGRP2 = SH2 // LANES
SELR = 158
SELC = 64
TROWS = CROWS // NS


def _nr_rsqrt(s):
  i = lax.bitcast_convert_type(s, jnp.int32)
  i = jnp.int32(0x5F3759DF) - (i >> 1)
  t = lax.bitcast_convert_type(i, jnp.float32)
  for _ in range(3):
    t = t * (jnp.float32(1.5) - jnp.float32(0.5) * s * t * t)
  return t


def _sc_gather_body(ox_hbm, oy_hbm, oz_hbm, m_hbm, src_hbm, dst_hbm,
                    msrc_hbm, c_hbm,
                    idxs_a, idxd_a, mrows_a, xyz_a,
                    idxs_b, idxd_b, mrows_b, xyz_b,
                    c_v, semm_a, semo_a, semm_b, semo_b):
  cid = lax.axis_index("c")
  sid = lax.axis_index("s")
  wid = sid * NC + cid
  wbase = wid * W1
  bufs_a = (idxs_a, idxd_a, mrows_a, xyz_a, semm_a, semo_a)
  bufs_b = (idxs_b, idxd_b, mrows_b, xyz_b, semm_b, semo_b)

  def o_copies(idxs_v, idxd_v, xyz_v, semo):
    return [
        pltpu.make_async_copy(ox_hbm.at[idxs_v], xyz_v.at[0], semo),
        pltpu.make_async_copy(oy_hbm.at[idxs_v], xyz_v.at[1], semo),
        pltpu.make_async_copy(oz_hbm.at[idxs_v], xyz_v.at[2], semo),
        pltpu.make_async_copy(ox_hbm.at[idxd_v], xyz_v.at[3], semo),
        pltpu.make_async_copy(oy_hbm.at[idxd_v], xyz_v.at[4], semo),
        pltpu.make_async_copy(oz_hbm.at[idxd_v], xyz_v.at[5], semo),
    ]

  def fire(k, bufs):
    idxs_v, idxd_v, mrows_v, xyz_v, semm, semo = bufs
    off = wbase + k * CH1
    pltpu.sync_copy(src_hbm.at[pl.ds(off, CH1)], idxs_v)
    pltpu.sync_copy(dst_hbm.at[pl.ds(off, CH1)], idxd_v)
    pltpu.async_copy(m_hbm.at[idxs_v], mrows_v, semm)
    for cp in o_copies(idxs_v, idxd_v, xyz_v, semo):
      cp.start()

  def drain(k, bufs):
    idxs_v, idxd_v, mrows_v, xyz_v, semm, semo = bufs
    off = wbase + k * CH1
    for cp in o_copies(idxs_v, idxd_v, xyz_v, semo):
      cp.wait()
    for g in range(CH1 // LANES):
      sl = pl.ds(g * LANES, LANES)
      x1, y1, z1 = xyz_v[0, sl], xyz_v[1, sl], xyz_v[2, sl]
      x2, y2, z2 = xyz_v[3, sl], xyz_v[4, sl], xyz_v[5, sl]
      dot = x1 * x2 + y1 * y2 + z1 * z2
      cx = y1 * z2 - z1 * y2
      cy = z1 * x2 - x1 * z2
      cz = x1 * y2 - y1 * x2
      s = dot * dot + cx * cx + cy * cy + cz * cz
      c_v[sl] = dot * _nr_rsqrt(s)
    pltpu.make_async_copy(m_hbm.at[idxs_v], mrows_v, semm).wait()
    pltpu.sync_copy(mrows_v, msrc_hbm.at[pl.ds(off, CH1)])
    pltpu.sync_copy(c_v, c_hbm.at[pl.ds(off, CH1)])

  fire(0, bufs_a)

  def pair(i, carry):
    k0 = i * 2
    fire(k0 + 1, bufs_b)
    drain(k0, bufs_a)
    fire(k0 + 2, bufs_a)
    drain(k0 + 1, bufs_b)
    return carry

  lax.fori_loop(0, NCH1 // 2, pair, 0)
  drain(NCH1 - 1, bufs_a)


def _make_sc_gather():
  mesh = plsc.VectorSubcoreMesh(
      core_axis_name="c", subcore_axis_name="s", num_cores=NC,
      num_subcores=NS)
  return pl.kernel(
      _sc_gather_body,
      out_type=[
          jax.ShapeDtypeStruct((LE, EMB), jnp.float32),
          jax.ShapeDtypeStruct((LE,), jnp.float32),
      ],
      mesh=mesh,
      compiler_params=pltpu.CompilerParams(needs_layout_passes=False),
      scratch_types=[
          pltpu.VMEM((CH1,), jnp.int32),
          pltpu.VMEM((CH1,), jnp.int32),
          pltpu.VMEM((CH1, EMB), jnp.float32),
          pltpu.VMEM((6, CH1), jnp.float32),
          pltpu.VMEM((CH1,), jnp.int32),
          pltpu.VMEM((CH1,), jnp.int32),
          pltpu.VMEM((CH1, EMB), jnp.float32),
          pltpu.VMEM((6, CH1), jnp.float32),
          pltpu.VMEM((CH1,), jnp.float32),
          pltpu.SemaphoreType.DMA,
          pltpu.SemaphoreType.DMA,
          pltpu.SemaphoreType.DMA,
          pltpu.SemaphoreType.DMA,
      ],
  )


def _sc_scatter_body(dst_hbm, xkj_hbm, zeros_hbm, mupd_hbm,
                     dstc_v, psel_v, rows0_v, rows1_v,
                     wselb0_v, dselb0_v, wselb1_v, dselb1_v,
                     sbuf, semg0, semg1, sema0, sema1):
  cid = lax.axis_index("c")
  sid = lax.axis_index("s")
  lane = lax.iota(jnp.int32, 16)
  c0 = lane * 0
  wbase = sid * SH2
  pltpu.sync_copy(dst_hbm.at[pl.ds(wbase, SH2)], dstc_v)

  def chunk(k, carry):
    chunk_id = k * NC + cid
    lo = chunk_id * CROWS
    pltpu.sync_copy(zeros_hbm.at[pl.ds(sid * TROWS, TROWS)],
                    sbuf.at[pl.ds(sid * TROWS, TROWS)])
    plsc.subcore_barrier()

    def scan(g, carry2):
      del g
      offm1, pwv, idxv = carry2
      dv = plsc.load_gather(dstc_v, [idxv])
      mask = (dv >> CSHIFT) == chunk_id
      mv = mask.astype(jnp.int32)
      pos = plsc.cumsum(mv) + offm1
      packed = pwv | (dv & (CROWS - 1))
      plsc.store_scatter(psel_v, [pos >> 7, pos & 127], packed, mask=mask)
      pc = plsc.all_reduce_population_count(mask)
      return (offm1 + pc, pwv + (LANES << CSHIFT), idxv + LANES)

    offm1, _, _ = lax.fori_loop(
        0, NGRP2, scan, (c0 - 1, lane << CSHIFT, lane))
    nsel = jnp.max(offm1) + 1
    nb = (nsel + (SELC - 1)) // SELC

    def unpack(b, wselb_v, dselb_v):
      for g in range(SELC // LANES):
        colv = lane + g * LANES
        fl = (b * SELC) + colv
        p = plsc.load_gather(psel_v, [fl >> 7, fl & 127])
        pm = fl >= nsel
        wv = jnp.where(pm, 0,
                       wbase + lax.shift_right_logical(p, CSHIFT))
        dv = jnp.where(pm, TRASH + (colv & 7), p & (CROWS - 1))
        plsc.store_scatter(wselb_v, [colv], wv)
        plsc.store_scatter(dselb_v, [colv], dv)

    def add0_copy():
      return pltpu.make_async_copy(rows0_v, sbuf.at[dselb0_v], sema0)

    def add1_copy():
      return pltpu.make_async_copy(rows1_v, sbuf.at[dselb1_v], sema1)

    def subpair(i, carry3):
      b0 = i * 2
      b1 = b0 + 1

      @pl.when(b0 < nb)
      def _fire0():
        @pl.when(b0 >= 2)
        def _reclaim0():
          add0_copy().wait()
        unpack(b0, wselb0_v, dselb0_v)
        pltpu.async_copy(xkj_hbm.at[wselb0_v], rows0_v, semg0)

      @pl.when(b1 < nb)
      def _fire1():
        @pl.when(b1 >= 2)
        def _reclaim1():
          add1_copy().wait()
        unpack(b1, wselb1_v, dselb1_v)
        pltpu.async_copy(xkj_hbm.at[wselb1_v], rows1_v, semg1)

      @pl.when(b0 < nb)
      def _drain0():
        pltpu.make_async_copy(xkj_hbm.at[wselb0_v], rows0_v, semg0).wait()
        pltpu.async_copy(rows0_v, sbuf.at[dselb0_v], sema0, add=True)

      @pl.when(b1 < nb)
      def _drain1():
        pltpu.make_async_copy(xkj_hbm.at[wselb1_v], rows1_v, semg1).wait()
        pltpu.async_copy(rows1_v, sbuf.at[dselb1_v], sema1, add=True)

      return carry3

    lax.fori_loop(0, (nb + 1) >> 1, subpair, 0)

    @pl.when(nb >= 1)
    def _final0():
      add0_copy().wait()

    @pl.when(nb >= 2)
    def _final1():
      add1_copy().wait()

    plsc.subcore_barrier()
    out0 = lo + sid * TROWS

    @pl.when(out0 + TROWS <= E)
    def _full_write():
      pltpu.sync_copy(sbuf.at[pl.ds(sid * TROWS, TROWS)],
                      mupd_hbm.at[pl.ds(out0, TROWS)])

    @pl.when(out0 == (E // TROWS) * TROWS)
    def _partial_write():
      pltpu.sync_copy(sbuf.at[pl.ds(sid * TROWS, E - (E // TROWS) * TROWS)],
                      mupd_hbm.at[pl.ds(out0, E - (E // TROWS) * TROWS)])

    return carry

  lax.fori_loop(0, NCHUNK2 // NC, chunk, 0)


def _make_sc_scatter():
  mesh = plsc.VectorSubcoreMesh(
      core_axis_name="c", subcore_axis_name="s", num_cores=NC,
      num_subcores=NS)
  return pl.kernel(
      _sc_scatter_body,
      out_type=jax.ShapeDtypeStruct((E, EMB), jnp.float32),
      mesh=mesh,
      compiler_params=pltpu.CompilerParams(needs_layout_passes=False),
      scratch_types=[
          pltpu.VMEM((SH2,), jnp.int32),
          pltpu.VMEM((SELR, 128), jnp.int32),
          pltpu.VMEM((SELC, EMB), jnp.float32),
          pltpu.VMEM((SELC, EMB), jnp.float32),
          pltpu.VMEM((SELC,), jnp.int32),
          pltpu.VMEM((SELC,), jnp.int32),
          pltpu.VMEM((SELC,), jnp.int32),
          pltpu.VMEM((SELC,), jnp.int32),
          pltpu.VMEM_SHARED((SB_ROWS, EMB), jnp.float32),
          pltpu.SemaphoreType.DMA,
          pltpu.SemaphoreType.DMA,
          pltpu.SemaphoreType.DMA,
          pltpu.SemaphoreType.DMA,
      ],
  )


BE = 640


def _w_body(rbf_ref, m_ref, wrbf_ref, wm_ref, bm_ref, w_ref):
  mt = jnp.dot(m_ref[...], wm_ref[...],
               preferred_element_type=jnp.float32) + bm_ref[...]
  mt = mt * jax.nn.sigmoid(mt)
  rp = jnp.dot(rbf_ref[...], wrbf_ref[...],
               preferred_element_type=jnp.float32)
  w_ref[...] = rp * mt


def _make_w_kernel():
  return pl.pallas_call(
      _w_body,
      grid=(E // BE,),
      in_specs=[
          pl.BlockSpec((BE, NRAD), lambda i: (i, 0)),
          pl.BlockSpec((BE, EMB), lambda i: (i, 0)),
          pl.BlockSpec((NRAD, EMB), lambda i: (0, 0)),
          pl.BlockSpec((EMB, EMB), lambda i: (0, 0)),
          pl.BlockSpec((EMB,), lambda i: (0,)),
      ],
      out_specs=pl.BlockSpec((BE, EMB), lambda i: (i, 0)),
      out_shape=jax.ShapeDtypeStruct((E, EMB), jnp.float32),
  )


BL = 512


def _xkj_body(c_ref, msrc_ref, wr_ref, w2_ref, xkj_ref):
  c = c_ref[...]
  ms = msrc_ref[...].astype(jnp.bfloat16)
  ts = [jnp.ones_like(c), c]
  for _ in range(2, NSPH):
    ts.append(jnp.float32(2.0) * c * ts[-1] - ts[-2])
  acc = None
  for j in range(NBIL):
    sj = ts[0] * wr_ref[0, j]
    for n in range(1, NSPH):
      sj = sj + ts[n] * wr_ref[n, j]
    sjb = sj.astype(jnp.bfloat16)
    p = jnp.dot(ms * sjb[:, None], w2_ref[j * EMB:(j + 1) * EMB, :],
                preferred_element_type=jnp.float32)
    acc = p if acc is None else acc + p
  xkj_ref[...] = acc


def _make_xkj_kernel():
  return pl.pallas_call(
      _xkj_body,
      grid=(LE // BL,),
      in_specs=[
          pl.BlockSpec((BL,), lambda i: (i,)),
          pl.BlockSpec((BL, EMB), lambda i: (i, 0)),
          pl.BlockSpec(memory_space=pltpu.SMEM),
          pl.BlockSpec((NBIL * EMB, EMB), lambda i: (0, 0)),
      ],
      out_specs=pl.BlockSpec((BL, EMB), lambda i: (i, 0)),
      out_shape=jax.ShapeDtypeStruct((LE, EMB), jnp.float32),
  )


def kernel(rbf, m, o, lg_src, lg_dst, W_rbf, W_sbf, W_m, b_m, W_bilin):
  ox, oy, oz = o[:, 0], o[:, 1], o[:, 2]
  wr = W_sbf.reshape(NSPH, NRAD, NBIL).sum(axis=1)
  w2 = W_bilin.transpose(1, 2, 0).reshape(NBIL * EMB, EMB).astype(jnp.bfloat16)
  zeros_c = jnp.zeros((CROWS, EMB), jnp.float32)

  m_src, c = _make_sc_gather()(ox, oy, oz, m, lg_src, lg_dst)
  w = _make_w_kernel()(rbf, m, W_rbf, W_m, b_m)
  x_kj = _make_xkj_kernel()(c, m_src, wr, w2)
  m_update = _make_sc_scatter()(lg_dst, x_kj, zeros_c)
  return (m_update, w)
